# baseline (speedup 1.0000x reference)
import functools

import jax
import jax.numpy as jnp
from jax import lax
from jax.experimental import pallas as pl
from jax.experimental.pallas import tpu as pltpu
from jax.experimental.pallas import tpu_sc as plsc

VOCAB = 100000
EMBED = 64
BATCH = 1024

_NC = 2
_NS = 16
_NW = _NC * _NS
_B_PER_W = BATCH // _NW

_VBLK = 2048


@functools.partial(
    pl.kernel,
    mesh=plsc.VectorSubcoreMesh(core_axis_name="c", subcore_axis_name="s"),
    out_type=jax.ShapeDtypeStruct((BATCH, 2 * EMBED), jnp.float32),
    scratch_types=[
        pltpu.VMEM((_B_PER_W,), jnp.int32),
        pltpu.VMEM((_B_PER_W, 2 * EMBED), jnp.float32),
        pltpu.SemaphoreType.DMA,
    ],
    compiler_params=pltpu.CompilerParams(use_tc_tiling_on_sc=True),
)
def _sc_gather(idx_hbm, table_hbm, out_hbm, idx_v, rows_v, sem):
    wid = lax.axis_index("s") * _NC + lax.axis_index("c")
    base = wid * _B_PER_W
    pltpu.sync_copy(idx_hbm.at[pl.ds(base, _B_PER_W)], idx_v)
    pltpu.async_copy(table_hbm.at[idx_v], rows_v, sem).wait()
    pltpu.sync_copy(rows_v, out_hbm.at[pl.ds(base, _B_PER_W)])


def _matmul_body(wt_ref, x2_ref, par_ref, b_ref, out_ref, x_s):
    k = pl.program_id(0)

    @pl.when(k == 0)
    def _():
        p = par_ref[...]
        x_s[...] = x2_ref[:, 0:EMBED] * (1.0 - p) + x2_ref[:, EMBED:] * p

    acc = lax.dot_general(
        wt_ref[...],
        x_s[...],
        (((0,), (1,)), ((), ())),
        preferred_element_type=jnp.float32,
    )
    bias = jax.lax.broadcast_in_dim(b_ref[0, :], (_VBLK, BATCH), (0,))
    out_ref[...] = acc + bias


def kernel(inputs_, emb, W, b):
    idx = inputs_.astype(jnp.int32)
    x2 = _sc_gather(idx // 2, emb.reshape(VOCAB // 2, 2 * EMBED))
    par = (idx & 1).astype(jnp.float32).reshape(BATCH, 1)

    out_t = pl.pallas_call(
        _matmul_body,
        grid=(pl.cdiv(VOCAB, _VBLK),),
        in_specs=[
            pl.BlockSpec((EMBED, _VBLK), lambda k: (0, k)),
            pl.BlockSpec((BATCH, 2 * EMBED), lambda k: (0, 0)),
            pl.BlockSpec((BATCH, 1), lambda k: (0, 0)),
            pl.BlockSpec((1, _VBLK), lambda k: (0, k)),
        ],
        out_specs=pl.BlockSpec((_VBLK, BATCH), lambda k: (k, 0)),
        out_shape=jax.ShapeDtypeStruct((VOCAB, BATCH), jnp.float32),
        scratch_shapes=[pltpu.VMEM((BATCH, EMBED), jnp.float32)],
    )(W.T, x2, par, b.reshape(1, VOCAB))
    return out_t.T


# speedup vs baseline: 1.0115x; 1.0115x over previous
import functools

import jax
import jax.numpy as jnp
from jax import lax
from jax.experimental import pallas as pl
from jax.experimental.pallas import tpu as pltpu
from jax.experimental.pallas import tpu_sc as plsc

VOCAB = 100000
EMBED = 64
BATCH = 1024

_NC = 2
_NS = 16
_NW = _NC * _NS
_B_PER_W = BATCH // _NW

_VBLK = 4096


@functools.partial(
    pl.kernel,
    mesh=plsc.VectorSubcoreMesh(core_axis_name="c", subcore_axis_name="s"),
    out_type=jax.ShapeDtypeStruct((BATCH, 2 * EMBED), jnp.float32),
    scratch_types=[
        pltpu.VMEM((_B_PER_W,), jnp.int32),
        pltpu.VMEM((_B_PER_W, 2 * EMBED), jnp.float32),
        pltpu.SemaphoreType.DMA,
    ],
    compiler_params=pltpu.CompilerParams(use_tc_tiling_on_sc=True),
)
def _sc_gather(idx_hbm, table_hbm, out_hbm, idx_v, rows_v, sem):
    wid = lax.axis_index("s") * _NC + lax.axis_index("c")
    base = wid * _B_PER_W
    pltpu.sync_copy(idx_hbm.at[pl.ds(base, _B_PER_W)], idx_v)
    pltpu.async_copy(table_hbm.at[idx_v], rows_v, sem).wait()
    pltpu.sync_copy(rows_v, out_hbm.at[pl.ds(base, _B_PER_W)])


def _matmul_body(wt_ref, x2_ref, par_ref, b_ref, out_ref, x_s):
    k = pl.program_id(0)

    @pl.when(k == 0)
    def _():
        p = par_ref[...]
        x_s[...] = x2_ref[:, 0:EMBED] * (1.0 - p) + x2_ref[:, EMBED:] * p

    acc = lax.dot_general(
        wt_ref[...],
        x_s[...],
        (((0,), (1,)), ((), ())),
        preferred_element_type=jnp.float32,
    )
    bias = jax.lax.broadcast_in_dim(b_ref[0, :], (_VBLK, BATCH), (0,))
    out_ref[...] = acc + bias


def kernel(inputs_, emb, W, b):
    idx = inputs_.astype(jnp.int32)
    x2 = _sc_gather(idx // 2, emb.reshape(VOCAB // 2, 2 * EMBED))
    par = (idx & 1).astype(jnp.float32).reshape(BATCH, 1)

    out_t = pl.pallas_call(
        _matmul_body,
        grid=(pl.cdiv(VOCAB, _VBLK),),
        in_specs=[
            pl.BlockSpec((EMBED, _VBLK), lambda k: (0, k)),
            pl.BlockSpec((BATCH, 2 * EMBED), lambda k: (0, 0)),
            pl.BlockSpec((BATCH, 1), lambda k: (0, 0)),
            pl.BlockSpec((1, _VBLK), lambda k: (0, k)),
        ],
        out_specs=pl.BlockSpec((_VBLK, BATCH), lambda k: (k, 0)),
        out_shape=jax.ShapeDtypeStruct((VOCAB, BATCH), jnp.float32),
        scratch_shapes=[pltpu.VMEM((BATCH, EMBED), jnp.float32)],
    )(W.T, x2, par, b.reshape(1, VOCAB))
    return out_t.T
